# Initial kernel scaffold; baseline (speedup 1.0000x reference)
import jax
import jax.numpy as jnp
from jax.experimental import pallas as pl


def kernel(decoder_input_ids, decoder_attention_mask, encoder_attention_mask, embed_weight):
    raise NotImplementedError("write your pallas kernel here")


# speedup vs baseline: 1.4873x; 1.4873x over previous
import functools

import jax
import jax.numpy as jnp
from jax import lax
from jax.experimental import pallas as pl
from jax.experimental.pallas import tpu as pltpu
from jax.experimental.pallas import tpu_sc as plsc

B = 4
S = 2048
S_ENC = 2048
D = 1024
HEADS = 16

NC = 2
NS = 16
NW = NC * NS
TOK = B * S
TPW = TOK // NW
CH = 64
NCHUNK = TPW // CH

RB = 512


@functools.partial(
    pl.kernel,
    out_type=jax.ShapeDtypeStruct((TOK, D), jnp.float32),
    mesh=plsc.VectorSubcoreMesh(core_axis_name="c", subcore_axis_name="s"),
    scratch_types=[
        pltpu.VMEM((NCHUNK, CH), jnp.int32),
        pltpu.VMEM((CH, D), jnp.float32),
        pltpu.SemaphoreType.DMA,
    ],
)
def _embed_gather(table_hbm, idx_hbm, out_hbm, idx_v, rows_v, sem):
    wid = lax.axis_index("s") * NC + lax.axis_index("c")
    base = wid * TPW
    for g in range(NCHUNK):
        off = base + g * CH
        pltpu.sync_copy(idx_hbm.at[pl.ds(off, CH)], idx_v.at[g])
        pltpu.async_copy(table_hbm.at[idx_v.at[g]], rows_v, sem).wait()
        pltpu.sync_copy(rows_v, out_hbm.at[pl.ds(off, CH)])


def _mask_body(dec_ref, enc_ref, ext_ref, encext_ref):
    r = pl.program_id(1)
    row = lax.broadcasted_iota(jnp.int32, (RB, S), 0) + r * RB
    col = lax.broadcasted_iota(jnp.int32, (RB, S), 1)
    causal = (col <= row).astype(jnp.float32)
    m = dec_ref[0]
    ext_ref[0, 0] = (1.0 - causal * m) * -10000.0
    encext_ref[...] = ((1.0 - enc_ref[...]) * -1e9).reshape(1, 1, 1, S_ENC)


_mask_call = pl.pallas_call(
    _mask_body,
    grid=(B, S // RB),
    in_specs=[
        pl.BlockSpec((1, 1, S), lambda b, r: (b, 0, 0)),
        pl.BlockSpec((1, 1, S_ENC), lambda b, r: (b, 0, 0)),
    ],
    out_specs=[
        pl.BlockSpec((1, 1, RB, S), lambda b, r: (b, 0, r, 0)),
        pl.BlockSpec((1, 1, 1, S_ENC), lambda b, r: (b, 0, 0, 0)),
    ],
    out_shape=[
        jax.ShapeDtypeStruct((B, 1, S, S), jnp.float32),
        jax.ShapeDtypeStruct((B, 1, 1, S_ENC), jnp.float32),
    ],
)


def kernel(decoder_input_ids, decoder_attention_mask, encoder_attention_mask, embed_weight):
    ids = decoder_input_ids.reshape(TOK).astype(jnp.int32)
    hidden = _embed_gather(embed_weight, ids)
    ext, encext = _mask_call(
        decoder_attention_mask.reshape(B, 1, S),
        encoder_attention_mask.reshape(B, 1, S_ENC),
    )
    bias = jnp.zeros((B, HEADS, S, 1), jnp.float32)
    return (hidden.reshape(B, S, D), encext, ext, bias)


# speedup vs baseline: 1.5476x; 1.0406x over previous
import functools

import jax
import jax.numpy as jnp
from jax import lax
from jax.experimental import pallas as pl
from jax.experimental.pallas import tpu as pltpu
from jax.experimental.pallas import tpu_sc as plsc

B = 4
S = 2048
S_ENC = 2048
D = 1024
HEADS = 16

NC = 2
NS = 16
NW = NC * NS
TPW = B * S // NW
SPW = S // TPW
CH = 32
NCHUNK = TPW // CH

RB = 512


@functools.partial(
    pl.kernel,
    out_type=jax.ShapeDtypeStruct((B, S, D), jnp.float32),
    mesh=plsc.VectorSubcoreMesh(core_axis_name="c", subcore_axis_name="s"),
    scratch_types=[
        pltpu.VMEM((TPW,), jnp.int32),
        pltpu.VMEM((CH, D), jnp.float32),
        pltpu.VMEM((CH, D), jnp.float32),
        pltpu.SemaphoreType.DMA,
        pltpu.SemaphoreType.DMA,
        pltpu.SemaphoreType.DMA,
        pltpu.SemaphoreType.DMA,
    ],
)
def _embed_gather(table_hbm, ids_hbm, out_hbm, idx_v, rows_a, rows_b,
                  g_sem_a, g_sem_b, o_sem_a, o_sem_b):
    wid = lax.axis_index("s") * NC + lax.axis_index("c")
    b = wid // SPW
    s0 = (wid % SPW) * TPW
    bufs = (rows_a, rows_b)
    g_sems = (g_sem_a, g_sem_b)
    o_sems = (o_sem_a, o_sem_b)

    pltpu.sync_copy(ids_hbm.at[b, pl.ds(s0, TPW)], idx_v)

    def gather_start(g):
        cp = pltpu.make_async_copy(
            table_hbm.at[idx_v.at[pl.ds(g * CH, CH)]], bufs[g % 2], g_sems[g % 2])
        cp.start()
        return cp

    def out_start(g):
        cp = pltpu.make_async_copy(
            bufs[g % 2], out_hbm.at[b, pl.ds(s0 + g * CH, CH)], o_sems[g % 2])
        cp.start()
        return cp

    pending_g = [gather_start(0)]
    pending_o = [None, None]
    for g in range(NCHUNK):
        if g + 1 < NCHUNK:
            if pending_o[(g + 1) % 2] is not None:
                pending_o[(g + 1) % 2].wait()
            pending_g.append(gather_start(g + 1))
        pending_g[g].wait()
        pending_o[g % 2] = out_start(g)
    pending_o[(NCHUNK - 1) % 2].wait()
    pending_o[NCHUNK % 2].wait()


def _mask_body(dec_ref, enc_ref, ext_ref, encext_ref):
    b = pl.program_id(0)
    r = pl.program_id(1)
    row = lax.broadcasted_iota(jnp.int32, (RB, S), 0) + r * RB
    col = lax.broadcasted_iota(jnp.int32, (RB, S), 1)
    m = dec_ref[pl.ds(b, 1), :]
    on_diag = -10000.0 * (1.0 - m)
    ext_ref[0, 0] = jnp.where(col <= row, on_diag, -10000.0)
    encext_ref[...] = ((1.0 - enc_ref[pl.ds(b, 1), :]) * -1e9).reshape(1, 1, 1, S_ENC)


_mask_call = pl.pallas_call(
    _mask_body,
    grid=(B, S // RB),
    in_specs=[
        pl.BlockSpec((B, S), lambda b, r: (0, 0)),
        pl.BlockSpec((B, S_ENC), lambda b, r: (0, 0)),
    ],
    out_specs=[
        pl.BlockSpec((1, 1, RB, S), lambda b, r: (b, 0, r, 0)),
        pl.BlockSpec((1, 1, 1, S_ENC), lambda b, r: (b, 0, 0, 0)),
    ],
    out_shape=[
        jax.ShapeDtypeStruct((B, 1, S, S), jnp.float32),
        jax.ShapeDtypeStruct((B, 1, 1, S_ENC), jnp.float32),
    ],
)


def kernel(decoder_input_ids, decoder_attention_mask, encoder_attention_mask, embed_weight):
    hidden = _embed_gather(embed_weight, decoder_input_ids)
    ext, encext = _mask_call(decoder_attention_mask, encoder_attention_mask)
    bias = jnp.zeros((B, HEADS, S, 1), jnp.float32)
    return (hidden, encext, ext, bias)
